# baseline (speedup 1.0000x reference)
import functools

import jax
import jax.numpy as jnp
from jax import lax
from jax.experimental import pallas as pl
from jax.experimental.pallas import tpu as pltpu
from jax.experimental.pallas import tpu_sc as plsc

_B = 4096
_LSEQ = 20
_EMB = 512
_DIM = 4
_NC, _NS, _LANES = 2, 16, 16
_NW = _NC * _NS
_ROWS_PER_W = _B // _NW
_GROUPS = _ROWS_PER_W // _LANES
_TAB_ROWS = 24


def _splat_u(v):
    return jnp.full((_LANES,), v, jnp.uint32)


def _splat_i(v):
    return jnp.full((_LANES,), v, jnp.int32)


@functools.partial(
    pl.kernel,
    out_type=jax.ShapeDtypeStruct((_B * _DIM,), jnp.float32),
    mesh=plsc.VectorSubcoreMesh(
        core_axis_name="c", subcore_axis_name="s",
        num_cores=_NC, num_subcores=_NS),
    compiler_params=pltpu.CompilerParams(needs_layout_passes=False),
    scratch_types=[
        pltpu.VMEM((_ROWS_PER_W * _LSEQ,), jnp.int32),
        pltpu.VMEM((_ROWS_PER_W * _LSEQ,), jnp.int32),
        pltpu.VMEM((_TAB_ROWS * _DIM,), jnp.float32),
        pltpu.VMEM((_ROWS_PER_W * _DIM,), jnp.float32),
        pltpu.SemaphoreType.DMA,
        pltpu.SemaphoreType.DMA,
        pltpu.SemaphoreType.DMA,
    ],
)
def _edit_distance_kernel(in1_hbm, in2_hbm, table_hbm, out_hbm,
                          in1_v, in2_v, table_v, out_v,
                          sem1, sem2, sem3):
    wid = lax.axis_index("s") * _NC + lax.axis_index("c")
    tok_base = wid * _ROWS_PER_W * _LSEQ
    out_base = wid * _ROWS_PER_W * _DIM
    c1 = pltpu.async_copy(
        in1_hbm.at[pl.ds(tok_base, _ROWS_PER_W * _LSEQ)], in1_v, sem1)
    c2 = pltpu.async_copy(
        in2_hbm.at[pl.ds(tok_base, _ROWS_PER_W * _LSEQ)], in2_v, sem2)
    c3 = pltpu.async_copy(
        table_hbm.at[pl.ds(0, _TAB_ROWS * _DIM)], table_v, sem3)
    c1.wait()
    c2.wait()
    c3.wait()

    lane = lax.iota(jnp.int32, _LANES)

    one = _splat_u(1)

    @plsc.parallel_loop(0, _GROUPS)
    def group_body(g):
        row_idx = g * _LANES + lane
        tok_idx = row_idx * _LSEQ
        b = [plsc.load_gather(in2_v, [tok_idx + _splat_i(j)])
             for j in range(_LSEQ)]

        def dp_row(i, row):
            ai = plsc.load_gather(in1_v, [tok_idx + i - 1])
            row = list(row)
            prev_diag = row[0]
            row[0] = jnp.full((_LANES,), i, jnp.int32).astype(jnp.uint32)
            for j in range(1, _LSEQ + 1):
                tmp = row[j]
                t = jnp.minimum(jnp.minimum(row[j], prev_diag), row[j - 1])
                row[j] = jnp.where(ai == b[j - 1], prev_diag, t + one)
                prev_diag = tmp
            return tuple(row)

        row = lax.fori_loop(
            1, _LSEQ + 1, dp_row,
            tuple(_splat_u(j) for j in range(_LSEQ + 1)))
        dist = jnp.minimum(
            row[_LSEQ], _splat_u(_TAB_ROWS - 1)).astype(jnp.int32)
        emb_idx = dist * _DIM
        out_idx = row_idx * _DIM
        for e in range(_DIM):
            vals = plsc.load_gather(table_v, [emb_idx + _splat_i(e)])
            plsc.store_scatter(out_v, [out_idx + _splat_i(e)], vals)

    pltpu.sync_copy(out_v, out_hbm.at[pl.ds(out_base, _ROWS_PER_W * _DIM)])


def kernel(input1, input2, embedding_table):
    out_flat = _edit_distance_kernel(
        input1.reshape(-1), input2.reshape(-1), embedding_table.reshape(-1))
    return out_flat.reshape(_B, _DIM)


# speedup vs baseline: 1.0042x; 1.0042x over previous
import functools

import jax
import jax.numpy as jnp
from jax import lax
from jax.experimental import pallas as pl
from jax.experimental.pallas import tpu as pltpu
from jax.experimental.pallas import tpu_sc as plsc

_B = 4096
_LSEQ = 20
_EMB = 512
_DIM = 4
_NC, _NS, _LANES = 2, 16, 16
_NW = _NC * _NS
_ROWS_PER_W = _B // _NW
_GROUPS = _ROWS_PER_W // _LANES
_TAB_ROWS = 24


def _splat_u(v):
    return jnp.full((_LANES,), v, jnp.uint32)


def _splat_i(v):
    return jnp.full((_LANES,), v, jnp.int32)


@functools.partial(
    pl.kernel,
    out_type=jax.ShapeDtypeStruct((_B * _DIM,), jnp.float32),
    mesh=plsc.VectorSubcoreMesh(
        core_axis_name="c", subcore_axis_name="s",
        num_cores=_NC, num_subcores=_NS),
    compiler_params=pltpu.CompilerParams(needs_layout_passes=False),
    scratch_types=[
        pltpu.VMEM((_ROWS_PER_W * _LSEQ,), jnp.int32),
        pltpu.VMEM((_ROWS_PER_W * _LSEQ,), jnp.int32),
        pltpu.VMEM((_TAB_ROWS * _DIM,), jnp.float32),
        pltpu.VMEM((_ROWS_PER_W * _DIM,), jnp.float32),
        pltpu.SemaphoreType.DMA,
        pltpu.SemaphoreType.DMA,
        pltpu.SemaphoreType.DMA,
    ],
)
def _edit_distance_kernel(in1_hbm, in2_hbm, table_hbm, out_hbm,
                          in1_v, in2_v, table_v, out_v,
                          sem1, sem2, sem3):
    wid = lax.axis_index("s") * _NC + lax.axis_index("c")
    tok_base = wid * _ROWS_PER_W * _LSEQ
    out_base = wid * _ROWS_PER_W * _DIM
    c1 = pltpu.async_copy(
        in1_hbm.at[pl.ds(tok_base, _ROWS_PER_W * _LSEQ)], in1_v, sem1)
    c2 = pltpu.async_copy(
        in2_hbm.at[pl.ds(tok_base, _ROWS_PER_W * _LSEQ)], in2_v, sem2)
    c3 = pltpu.async_copy(
        table_hbm.at[pl.ds(0, _TAB_ROWS * _DIM)], table_v, sem3)
    c1.wait()
    c2.wait()
    c3.wait()

    lane = lax.iota(jnp.int32, _LANES)

    one = _splat_u(1)

    @plsc.parallel_loop(0, _GROUPS)
    def group_body(g):
        row_idx = g * _LANES + lane
        tok_idx = row_idx * _LSEQ
        b = [plsc.load_gather(in2_v, [tok_idx + _splat_i(j)])
             for j in range(_LSEQ)]

        def dp_row(i, row):
            ai = plsc.load_gather(in1_v, [tok_idx + i - 1])
            row = list(row)
            prev_diag = row[0]
            row[0] = jnp.full((_LANES,), i, jnp.int32).astype(jnp.uint32)
            for j in range(1, _LSEQ + 1):
                tmp = row[j]
                t = jnp.minimum(jnp.minimum(row[j], prev_diag), row[j - 1])
                row[j] = jnp.where(ai == b[j - 1], prev_diag, t + one)
                prev_diag = tmp
            return tuple(row)

        row = lax.fori_loop(
            1, _LSEQ + 1, dp_row,
            tuple(_splat_u(j) for j in range(_LSEQ + 1)), unroll=2)
        dist = jnp.minimum(
            row[_LSEQ], _splat_u(_TAB_ROWS - 1)).astype(jnp.int32)
        emb_idx = dist * _DIM
        out_idx = row_idx * _DIM
        for e in range(_DIM):
            vals = plsc.load_gather(table_v, [emb_idx + _splat_i(e)])
            plsc.store_scatter(out_v, [out_idx + _splat_i(e)], vals)

    pltpu.sync_copy(out_v, out_hbm.at[pl.ds(out_base, _ROWS_PER_W * _DIM)])


def kernel(input1, input2, embedding_table):
    out_flat = _edit_distance_kernel(
        input1.reshape(-1), input2.reshape(-1), embedding_table.reshape(-1))
    return out_flat.reshape(_B, _DIM)
